# baseline (speedup 1.0000x reference)
import functools

import jax
import jax.numpy as jnp
from jax import lax
from jax.experimental import pallas as pl
from jax.experimental.pallas import tpu as pltpu
from jax.experimental.pallas import tpu_sc as plsc

N = 10000
E = 320000
D = 128
NC = 2
NS = 16
NW = NC * NS
K = 128
GCH = E // K
NCH = GCH // NW
NCH_MAX = NCH + 2
DCH = GCH // NS
DCH_MAX = DCH + 4
SEG = 640
N_PAD = NS * SEG
HSEG = SEG // 2

_MESH = plsc.VectorSubcoreMesh(core_axis_name="c", subcore_axis_name="s")


BK = 10
DCH_B = 160
DCH_T = GCH - (NS - 1) * DCH_B


def _dinv_body(edge_hbm, out_hbm, didx, ones, ssem, dv, bc, acc):
    c = lax.axis_index("c")
    s = lax.axis_index("s")

    @pl.loop(0, SEG // 16)
    def _(i):
        ones[pl.ds(i * 16, 16)] = jnp.zeros((16,), jnp.float32)

    pltpu.sync_copy(ones, acc.at[pl.ds(s * SEG, SEG)])

    @pl.loop(0, K // 16)
    def _(i):
        ones[pl.ds(i * 16, 16)] = jnp.ones((16,), jnp.float32)

    nch = jnp.where(s < NS - 1, DCH_B, GCH - (NS - 1) * DCH_B)

    @pl.when(s < NS - 1)
    def _():
        pltpu.sync_copy(edge_hbm.at[1].at[pl.ds(s * DCH_B, DCH_B)], didx)

    @pl.when(s == NS - 1)
    def _():
        pltpu.sync_copy(edge_hbm.at[1].at[pl.ds((NS - 1) * DCH_B, DCH_T)],
                        didx.at[pl.ds(0, DCH_T)])

    plsc.subcore_barrier()

    ones_k = ones.at[pl.ds(0, K)]

    @pl.loop(0, nch, step=BK)
    def _(j):
        for t in range(BK):
            pltpu.async_copy(ones_k, acc.at[didx.at[j + t]], ssem, add=True)
        for t in range(BK):
            pltpu.make_async_copy(ones_k, acc.at[didx.at[j + t]], ssem).wait()

    plsc.subcore_barrier()

    base = c * (N_PAD // 2) + s * HSEG
    pltpu.sync_copy(acc.at[pl.ds(base, HSEG)], dv)

    @pl.loop(0, HSEG // 16)
    def _(i):
        v = dv[pl.ds(i * 16, 16)]
        bits = plsc.bitcast(v, jnp.int32)
        bits = jnp.int32(0x5F3759DF) - (bits >> 1)
        y = plsc.bitcast(bits, jnp.float32)
        for _ in range(3):
            y = y * (1.5 - 0.5 * v * y * y)
        dv[pl.ds(i * 16, 16)] = jnp.where(v > 0.5, y, 0.0)

    @pl.loop(0, HSEG)
    def _(r):
        idxv = jnp.full((16,), r, dtype=jnp.int32)
        row = plsc.load_gather(dv, [idxv])
        for jj in range(D // 16):
            bc[r, pl.ds(jj * 16, 16)] = row

    pltpu.sync_copy(bc, out_hbm.at[pl.ds(base, HSEG)])


_dinv_call = functools.partial(
    pl.kernel,
    out_type=jax.ShapeDtypeStruct((N_PAD, D), jnp.float32),
    mesh=_MESH,
    scratch_types=[
        pltpu.VMEM((DCH_B, K), jnp.int32),
        pltpu.VMEM((SEG,), jnp.float32),
        pltpu.SemaphoreType.DMA,
        pltpu.VMEM((HSEG,), jnp.float32),
        pltpu.VMEM((HSEG, D), jnp.float32),
        pltpu.VMEM_SHARED((N_PAD,), jnp.float32),
    ],
    compiler_params=pltpu.CompilerParams(needs_layout_passes=False),
)(_dinv_body)


def _main_body(xs_hbm, edge_hbm, out_hbm,
               sibuf, sisem, dibuf, disem, rbuf, rsem, ssem, acc):
    c = lax.axis_index("c")
    s = lax.axis_index("s")
    wid = s * NC + c
    nch = NCH + jnp.where(wid < GCH - NCH * NW, 1, 0)

    def chunk(jj):
        return jnp.where(jj < NCH, wid * NCH + jj, NCH * NW + wid)

    @pl.loop(0, K)
    def _(i):
        for jj in range(D // 16):
            rbuf[0][i, pl.ds(jj * 16, 16)] = jnp.zeros((16,), jnp.float32)

    for k in range(SEG // K):
        pltpu.sync_copy(rbuf[0], acc.at[pl.ds(s * SEG + k * K, K)])

    plsc.subcore_barrier()

    def start_idx(jj, slot):
        g = chunk(jj)
        pltpu.async_copy(edge_hbm.at[0, pl.ds(g * K, K)], sibuf[slot],
                         sisem[slot])
        pltpu.async_copy(edge_hbm.at[1, pl.ds(g * K, K)], dibuf[slot],
                         disem[slot])

    def wait_sidx(slot):
        pltpu.make_async_copy(edge_hbm.at[0, pl.ds(0, K)], sibuf[slot],
                              sisem[slot]).wait()

    def wait_didx(slot):
        pltpu.make_async_copy(edge_hbm.at[1, pl.ds(0, K)], dibuf[slot],
                              disem[slot]).wait()

    def start_rows(slot, rb):
        pltpu.async_copy(xs_hbm.at[sibuf[slot]], rbuf[rb], rsem[rb])

    def wait_rows(slot, rb):
        pltpu.make_async_copy(xs_hbm.at[sibuf[slot]], rbuf[rb],
                              rsem[rb]).wait()

    def wait_scat(rb):
        pltpu.make_async_copy(rbuf[rb], acc.at[dibuf[0]],
                              ssem[rb]).wait()

    for t in range(4):
        start_idx(t, t)
    wait_sidx(0)
    start_rows(0, 0)

    @pl.loop(0, NCH_MAX, step=4)
    def _(j):
        for b in range(4):
            jj = j + b
            rb = b % 2

            @pl.when(jj < nch)
            def _():
                wait_rows(b, rb)
                wait_didx(b)
                pltpu.async_copy(rbuf[rb], acc.at[dibuf[b]],
                                 ssem[rb], add=True)

            @pl.when(jj + 4 < nch)
            def _():
                start_idx(jj + 4, b)

            @pl.when((jj + 1 < nch) & (jj >= 1))
            def _():
                wait_scat(1 - rb)

            @pl.when(jj + 1 < nch)
            def _():
                wait_sidx((b + 1) % 4)
                start_rows((b + 1) % 4, 1 - rb)

    wait_scat(0)
    wait_scat(1)

    plsc.subcore_barrier()
    pltpu.sync_copy(acc.at[pl.ds(s * SEG, SEG)],
                    out_hbm.at[c].at[pl.ds(s * SEG, SEG)])


_main_call = functools.partial(
    pl.kernel,
    out_type=jax.ShapeDtypeStruct((NC, N_PAD, D), jnp.float32),
    mesh=_MESH,
    scratch_types=[
        [pltpu.VMEM((K,), jnp.int32)] * 4,
        [pltpu.SemaphoreType.DMA] * 4,
        [pltpu.VMEM((K,), jnp.int32)] * 4,
        [pltpu.SemaphoreType.DMA] * 4,
        [pltpu.VMEM((K, D), jnp.float32)] * 2,
        [pltpu.SemaphoreType.DMA] * 2,
        [pltpu.SemaphoreType.DMA] * 2,
        pltpu.VMEM_SHARED((N_PAD, D), jnp.float32),
    ],
)(_main_body)


def _prescale_body(x_ref, dinv_ref, xs_ref):
    xs_ref[...] = x_ref[...] * dinv_ref[...]


def _postscale_body(q_ref, dinv_ref, o_ref):
    o_ref[...] = (q_ref[0] + q_ref[1]) * dinv_ref[...]


_RB = 1000
_G = N // _RB

_prescale = pl.pallas_call(
    _prescale_body,
    grid=(_G,),
    in_specs=[
        pl.BlockSpec((_RB, D), lambda i: (i, 0)),
        pl.BlockSpec((_RB, D), lambda i: (i, 0)),
    ],
    out_specs=pl.BlockSpec((_RB, D), lambda i: (i, 0)),
    out_shape=jax.ShapeDtypeStruct((N, D), jnp.float32),
)

_postscale = pl.pallas_call(
    _postscale_body,
    grid=(_G,),
    in_specs=[
        pl.BlockSpec((NC, _RB, D), lambda i: (0, i, 0)),
        pl.BlockSpec((_RB, D), lambda i: (i, 0)),
    ],
    out_specs=pl.BlockSpec((_RB, D), lambda i: (i, 0)),
    out_shape=jax.ShapeDtypeStruct((N, D), jnp.float32),
)


@jax.jit
def kernel(x, edge_index):
    edge3 = edge_index.reshape(2, GCH, K)
    dinv = _dinv_call(edge3)
    xs = _prescale(x, dinv)
    q = _main_call(xs, edge_index)
    return _postscale(q, dinv)


# speedup vs baseline: 1.1460x; 1.1460x over previous
import functools

import jax
import jax.numpy as jnp
from jax import lax
from jax.experimental import pallas as pl
from jax.experimental.pallas import tpu as pltpu
from jax.experimental.pallas import tpu_sc as plsc

N = 10000
E = 320000
D = 128
NC = 2
NS = 16
NW = NC * NS
K = 128
GCH = E // K
NCH = GCH // NW
NCH_MAX = NCH + 2
DCH = GCH // NS
DCH_MAX = DCH + 4
SEG = 640
N_PAD = NS * SEG
HSEG = SEG // 2

_MESH = plsc.VectorSubcoreMesh(core_axis_name="c", subcore_axis_name="s")


BK = 10
DCH_B = 160
DCH_T = GCH - (NS - 1) * DCH_B


def _dinv_body(edge_hbm, out_hbm, didx, ones, ssem, dv, bc, acc):
    c = lax.axis_index("c")
    s = lax.axis_index("s")

    @pl.loop(0, SEG // 16)
    def _(i):
        ones[pl.ds(i * 16, 16)] = jnp.zeros((16,), jnp.float32)

    pltpu.sync_copy(ones, acc.at[pl.ds(s * SEG, SEG)])

    @pl.loop(0, K // 16)
    def _(i):
        ones[pl.ds(i * 16, 16)] = jnp.ones((16,), jnp.float32)

    nch = jnp.where(s < NS - 1, DCH_B, GCH - (NS - 1) * DCH_B)

    @pl.when(s < NS - 1)
    def _():
        pltpu.sync_copy(edge_hbm.at[1].at[pl.ds(s * DCH_B, DCH_B)], didx)

    @pl.when(s == NS - 1)
    def _():
        pltpu.sync_copy(edge_hbm.at[1].at[pl.ds((NS - 1) * DCH_B, DCH_T)],
                        didx.at[pl.ds(0, DCH_T)])

    plsc.subcore_barrier()

    ones_k = ones.at[pl.ds(0, K)]

    @pl.loop(0, nch, step=BK)
    def _(j):
        for t in range(BK):
            pltpu.async_copy(ones_k, acc.at[didx.at[j + t]], ssem, add=True)
        for t in range(BK):
            pltpu.make_async_copy(ones_k, acc.at[didx.at[j + t]], ssem).wait()

    plsc.subcore_barrier()

    base = c * (N_PAD // 2) + s * HSEG
    pltpu.sync_copy(acc.at[pl.ds(base, HSEG)], dv)

    @pl.loop(0, HSEG // 16)
    def _(i):
        v = dv[pl.ds(i * 16, 16)]
        bits = plsc.bitcast(v, jnp.int32)
        bits = jnp.int32(0x5F3759DF) - (bits >> 1)
        y = plsc.bitcast(bits, jnp.float32)
        for _ in range(3):
            y = y * (1.5 - 0.5 * v * y * y)
        dv[pl.ds(i * 16, 16)] = jnp.where(v > 0.5, y, 0.0)

    @pl.loop(0, HSEG)
    def _(r):
        idxv = jnp.full((16,), r, dtype=jnp.int32)
        row = plsc.load_gather(dv, [idxv])
        for jj in range(D // 16):
            bc[r, pl.ds(jj * 16, 16)] = row

    pltpu.sync_copy(bc, out_hbm.at[pl.ds(base, HSEG)])


_dinv_call = functools.partial(
    pl.kernel,
    out_type=jax.ShapeDtypeStruct((N_PAD, D), jnp.float32),
    mesh=_MESH,
    scratch_types=[
        pltpu.VMEM((DCH_B, K), jnp.int32),
        pltpu.VMEM((SEG,), jnp.float32),
        pltpu.SemaphoreType.DMA,
        pltpu.VMEM((HSEG,), jnp.float32),
        pltpu.VMEM((HSEG, D), jnp.float32),
        pltpu.VMEM_SHARED((N_PAD,), jnp.float32),
    ],
    compiler_params=pltpu.CompilerParams(needs_layout_passes=False),
)(_dinv_body)


def _main_body(xs_hbm, edge_hbm, out_hbm,
               sibuf, sisem, dibuf, disem, rbuf, rsem, acc):
    c = lax.axis_index("c")
    s = lax.axis_index("s")
    wid = s * NC + c
    nch = NCH + jnp.where(wid < GCH - NCH * NW, 1, 0)

    def chunk(jj):
        return jnp.where(jj < NCH, wid * NCH + jj, NCH * NW + wid)

    @pl.loop(0, K)
    def _(i):
        for jj in range(D // 16):
            rbuf[0][i, pl.ds(jj * 16, 16)] = jnp.zeros((16,), jnp.float32)

    for k in range(SEG // K):
        pltpu.sync_copy(rbuf[0], acc.at[pl.ds(s * SEG + k * K, K)])

    plsc.subcore_barrier()

    def start_idx(jj, slot):
        g = chunk(jj)
        pltpu.async_copy(edge_hbm.at[0, pl.ds(g * K, K)], sibuf[slot],
                         sisem[slot])
        pltpu.async_copy(edge_hbm.at[1, pl.ds(g * K, K)], dibuf[slot],
                         disem[slot])

    def wait_sidx(slot):
        pltpu.make_async_copy(edge_hbm.at[0, pl.ds(0, K)], sibuf[slot],
                              sisem[slot]).wait()

    def wait_didx(slot):
        pltpu.make_async_copy(edge_hbm.at[1, pl.ds(0, K)], dibuf[slot],
                              disem[slot]).wait()

    def start_rows(slot, rb):
        pltpu.async_copy(xs_hbm.at[sibuf[slot]], rbuf[rb], rsem[rb])

    def wait_rows(slot, rb):
        pltpu.make_async_copy(xs_hbm.at[sibuf[slot]], rbuf[rb],
                              rsem[rb]).wait()

    for t in range(4):
        start_idx(t, t)
    for t in range(2):
        wait_sidx(t)
        start_rows(t, t)

    @pl.loop(0, NCH_MAX, step=4)
    def _(j):
        for b in range(4):
            jj = j + b
            rb = b % 2

            @pl.when(jj < nch)
            def _():
                wait_rows(b, rb)
                wait_didx(b)
                pltpu.sync_copy(rbuf[rb], acc.at[dibuf[b]], add=True)

            @pl.when(jj + 4 < nch)
            def _():
                start_idx(jj + 4, b)

            @pl.when(jj + 2 < nch)
            def _():
                wait_sidx((b + 2) % 4)
                start_rows((b + 2) % 4, rb)

    plsc.subcore_barrier()
    pltpu.sync_copy(acc.at[pl.ds(s * SEG, SEG)],
                    out_hbm.at[c].at[pl.ds(s * SEG, SEG)])


_main_call = functools.partial(
    pl.kernel,
    out_type=jax.ShapeDtypeStruct((NC, N_PAD, D), jnp.float32),
    mesh=_MESH,
    scratch_types=[
        [pltpu.VMEM((K,), jnp.int32)] * 4,
        [pltpu.SemaphoreType.DMA] * 4,
        [pltpu.VMEM((K,), jnp.int32)] * 4,
        [pltpu.SemaphoreType.DMA] * 4,
        [pltpu.VMEM((K, D), jnp.float32)] * 2,
        [pltpu.SemaphoreType.DMA] * 2,
        pltpu.VMEM_SHARED((N_PAD, D), jnp.float32),
    ],
)(_main_body)


def _prescale_body(x_ref, dinv_ref, xs_ref):
    xs_ref[...] = x_ref[...] * dinv_ref[...]


def _postscale_body(q_ref, dinv_ref, o_ref):
    o_ref[...] = (q_ref[0] + q_ref[1]) * dinv_ref[...]


_RB = 1000
_G = N // _RB

_prescale = pl.pallas_call(
    _prescale_body,
    grid=(_G,),
    in_specs=[
        pl.BlockSpec((_RB, D), lambda i: (i, 0)),
        pl.BlockSpec((_RB, D), lambda i: (i, 0)),
    ],
    out_specs=pl.BlockSpec((_RB, D), lambda i: (i, 0)),
    out_shape=jax.ShapeDtypeStruct((N, D), jnp.float32),
)

_postscale = pl.pallas_call(
    _postscale_body,
    grid=(_G,),
    in_specs=[
        pl.BlockSpec((NC, _RB, D), lambda i: (0, i, 0)),
        pl.BlockSpec((_RB, D), lambda i: (i, 0)),
    ],
    out_specs=pl.BlockSpec((_RB, D), lambda i: (i, 0)),
    out_shape=jax.ShapeDtypeStruct((N, D), jnp.float32),
)


@jax.jit
def kernel(x, edge_index):
    edge3 = edge_index.reshape(2, GCH, K)
    dinv = _dinv_call(edge3)
    xs = _prescale(x, dinv)
    q = _main_call(xs, edge_index)
    return _postscale(q, dinv)


# speedup vs baseline: 1.1471x; 1.0010x over previous
import functools

import jax
import jax.numpy as jnp
from jax import lax
from jax.experimental import pallas as pl
from jax.experimental.pallas import tpu as pltpu
from jax.experimental.pallas import tpu_sc as plsc

N = 10000
E = 320000
D = 128
NC = 2
NS = 16
NW = NC * NS
K = 128
GCH = E // K
NCH = GCH // NW
NCH_MAX = NCH + 2
DCH = GCH // NS
DCH_MAX = DCH + 4
SEG = 640
N_PAD = NS * SEG
HSEG = SEG // 2

_MESH = plsc.VectorSubcoreMesh(core_axis_name="c", subcore_axis_name="s")


BK = 10
DCH_B = 160
DCH_T = GCH - (NS - 1) * DCH_B


def _dinv_body(edge_hbm, out_hbm, didx, ones, ssem, dv, bc, acc):
    c = lax.axis_index("c")
    s = lax.axis_index("s")

    @pl.loop(0, SEG // 16)
    def _(i):
        ones[pl.ds(i * 16, 16)] = jnp.zeros((16,), jnp.float32)

    pltpu.sync_copy(ones, acc.at[pl.ds(s * SEG, SEG)])

    @pl.loop(0, K // 16)
    def _(i):
        ones[pl.ds(i * 16, 16)] = jnp.ones((16,), jnp.float32)

    nch = jnp.where(s < NS - 1, DCH_B, GCH - (NS - 1) * DCH_B)

    @pl.when(s < NS - 1)
    def _():
        pltpu.sync_copy(edge_hbm.at[1].at[pl.ds(s * DCH_B, DCH_B)], didx)

    @pl.when(s == NS - 1)
    def _():
        pltpu.sync_copy(edge_hbm.at[1].at[pl.ds((NS - 1) * DCH_B, DCH_T)],
                        didx.at[pl.ds(0, DCH_T)])

    plsc.subcore_barrier()

    ones_k = ones.at[pl.ds(0, K)]

    @pl.loop(0, nch, step=BK)
    def _(j):
        for t in range(BK):
            pltpu.async_copy(ones_k, acc.at[didx.at[j + t]], ssem, add=True)
        for t in range(BK):
            pltpu.make_async_copy(ones_k, acc.at[didx.at[j + t]], ssem).wait()

    plsc.subcore_barrier()

    base = c * (N_PAD // 2) + s * HSEG
    pltpu.sync_copy(acc.at[pl.ds(base, HSEG)], dv)

    @pl.loop(0, HSEG // 16)
    def _(i):
        v = dv[pl.ds(i * 16, 16)]
        bits = plsc.bitcast(v, jnp.int32)
        bits = jnp.int32(0x5F3759DF) - (bits >> 1)
        y = plsc.bitcast(bits, jnp.float32)
        for _ in range(3):
            y = y * (1.5 - 0.5 * v * y * y)
        dv[pl.ds(i * 16, 16)] = jnp.where(v > 0.5, y, 0.0)

    @pl.loop(0, HSEG)
    def _(r):
        idxv = jnp.full((16,), r, dtype=jnp.int32)
        row = plsc.load_gather(dv, [idxv])
        for jj in range(D // 16):
            bc[r, pl.ds(jj * 16, 16)] = row

    pltpu.sync_copy(bc, out_hbm.at[pl.ds(base, HSEG)])


_dinv_call = functools.partial(
    pl.kernel,
    out_type=jax.ShapeDtypeStruct((N_PAD, D), jnp.float32),
    mesh=_MESH,
    scratch_types=[
        pltpu.VMEM((DCH_B, K), jnp.int32),
        pltpu.VMEM((SEG,), jnp.float32),
        pltpu.SemaphoreType.DMA,
        pltpu.VMEM((HSEG,), jnp.float32),
        pltpu.VMEM((HSEG, D), jnp.float32),
        pltpu.VMEM_SHARED((N_PAD,), jnp.float32),
    ],
    compiler_params=pltpu.CompilerParams(needs_layout_passes=False),
)(_dinv_body)


H = K // 2


def _main_body(xs_hbm, edge_hbm, out_hbm,
               sibuf, sisem, dibuf, disem, rbuf, rsem, ssem, acc):
    c = lax.axis_index("c")
    s = lax.axis_index("s")
    wid = s * NC + c
    nch = NCH + jnp.where(wid < GCH - NCH * NW, 1, 0)

    def chunk(jj):
        return jnp.where(jj < NCH, wid * NCH + jj, NCH * NW + wid)

    @pl.loop(0, K)
    def _(i):
        for jj in range(D // 16):
            rbuf[0][i, pl.ds(jj * 16, 16)] = jnp.zeros((16,), jnp.float32)

    for k in range(SEG // K):
        pltpu.sync_copy(rbuf[0], acc.at[pl.ds(s * SEG + k * K, K)])

    plsc.subcore_barrier()

    def start_idx(jj, slot):
        g = chunk(jj)
        pltpu.async_copy(edge_hbm.at[0, pl.ds(g * K, K)], sibuf[slot],
                         sisem[slot])
        pltpu.async_copy(edge_hbm.at[1, pl.ds(g * K, H)], dibuf[slot].at[0],
                         disem[slot])
        pltpu.async_copy(edge_hbm.at[1, pl.ds(g * K + H, H)],
                         dibuf[slot].at[1], disem[slot])

    def wait_sidx(slot):
        pltpu.make_async_copy(edge_hbm.at[0, pl.ds(0, K)], sibuf[slot],
                              sisem[slot]).wait()

    def wait_didx(slot):
        for h in range(2):
            pltpu.make_async_copy(edge_hbm.at[1, pl.ds(0, H)],
                                  dibuf[slot].at[h], disem[slot]).wait()

    def start_rows(slot, rb):
        pltpu.async_copy(xs_hbm.at[sibuf[slot]], rbuf[rb], rsem[rb])

    def wait_rows(slot, rb):
        pltpu.make_async_copy(xs_hbm.at[sibuf[slot]], rbuf[rb],
                              rsem[rb]).wait()

    for t in range(4):
        start_idx(t, t)
    for t in range(2):
        wait_sidx(t)
        start_rows(t, t)

    @pl.loop(0, NCH_MAX, step=4)
    def _(j):
        for b in range(4):
            jj = j + b
            rb = b % 2

            @pl.when(jj < nch)
            def _():
                wait_rows(b, rb)
                wait_didx(b)
                for h in range(2):
                    pltpu.async_copy(rbuf[rb].at[pl.ds(h * H, H)],
                                     acc.at[dibuf[b].at[h]], ssem[h],
                                     add=True)
                for h in range(2):
                    pltpu.make_async_copy(rbuf[rb].at[pl.ds(h * H, H)],
                                          acc.at[dibuf[b].at[h]],
                                          ssem[h]).wait()

            @pl.when(jj + 4 < nch)
            def _():
                start_idx(jj + 4, b)

            @pl.when(jj + 2 < nch)
            def _():
                wait_sidx((b + 2) % 4)
                start_rows((b + 2) % 4, rb)

    plsc.subcore_barrier()
    pltpu.sync_copy(acc.at[pl.ds(s * SEG, SEG)],
                    out_hbm.at[c].at[pl.ds(s * SEG, SEG)])


_main_call = functools.partial(
    pl.kernel,
    out_type=jax.ShapeDtypeStruct((NC, N_PAD, D), jnp.float32),
    mesh=_MESH,
    scratch_types=[
        [pltpu.VMEM((K,), jnp.int32)] * 4,
        [pltpu.SemaphoreType.DMA] * 4,
        [pltpu.VMEM((2, H), jnp.int32)] * 4,
        [pltpu.SemaphoreType.DMA] * 4,
        [pltpu.VMEM((K, D), jnp.float32)] * 2,
        [pltpu.SemaphoreType.DMA] * 2,
        [pltpu.SemaphoreType.DMA] * 2,
        pltpu.VMEM_SHARED((N_PAD, D), jnp.float32),
    ],
)(_main_body)


def _prescale_body(x_ref, dinv_ref, xs_ref):
    xs_ref[...] = x_ref[...] * dinv_ref[...]


def _postscale_body(q_ref, dinv_ref, o_ref):
    o_ref[...] = (q_ref[0] + q_ref[1]) * dinv_ref[...]


_RB = 1000
_G = N // _RB

_prescale = pl.pallas_call(
    _prescale_body,
    grid=(_G,),
    in_specs=[
        pl.BlockSpec((_RB, D), lambda i: (i, 0)),
        pl.BlockSpec((_RB, D), lambda i: (i, 0)),
    ],
    out_specs=pl.BlockSpec((_RB, D), lambda i: (i, 0)),
    out_shape=jax.ShapeDtypeStruct((N, D), jnp.float32),
)

_postscale = pl.pallas_call(
    _postscale_body,
    grid=(_G,),
    in_specs=[
        pl.BlockSpec((NC, _RB, D), lambda i: (0, i, 0)),
        pl.BlockSpec((_RB, D), lambda i: (i, 0)),
    ],
    out_specs=pl.BlockSpec((_RB, D), lambda i: (i, 0)),
    out_shape=jax.ShapeDtypeStruct((N, D), jnp.float32),
)


@jax.jit
def kernel(x, edge_index):
    edge3 = edge_index.reshape(2, GCH, K)
    dinv = _dinv_call(edge3)
    xs = _prescale(x, dinv)
    q = _main_call(xs, edge_index)
    return _postscale(q, dinv)
